# Initial kernel scaffold; baseline (speedup 1.0000x reference)
import jax
import jax.numpy as jnp
from jax.experimental import pallas as pl


def kernel(x, edge_index, W1, b1, W2, b2, Wo, bo):
    raise NotImplementedError("write your pallas kernel here")


# speedup vs baseline: 3.0366x; 3.0366x over previous
import functools
import jax
import jax.numpy as jnp
from jax.experimental import pallas as pl
from jax.experimental.pallas import tpu as pltpu

N_NODES = 10000
F = 128
OUT_DIM = 64
ROW_BLK = 1000
N_BLK = N_NODES // ROW_BLK


def _tc1_body(x_ref, w1_ref, degp_ref, h1_ref, g1_ref, dinv_ref):
    deg = 1.0 + degp_ref[0] + degp_ref[1]
    dinv = jax.lax.rsqrt(deg)
    h1 = jnp.dot(x_ref[...], w1_ref[...], preferred_element_type=jnp.float32)
    h1_ref[...] = h1
    g1_ref[...] = h1 * dinv
    dinv_ref[...] = dinv


def _tc1(x, W1, degp):
    return pl.pallas_call(
        _tc1_body,
        grid=(N_BLK,),
        in_specs=[
            pl.BlockSpec((ROW_BLK, F), lambda i: (i, 0)),
            pl.BlockSpec((F, F), lambda i: (0, 0)),
            pl.BlockSpec((2, ROW_BLK, 1), lambda i: (0, i, 0)),
        ],
        out_specs=[
            pl.BlockSpec((ROW_BLK, F), lambda i: (i, 0)),
            pl.BlockSpec((ROW_BLK, F), lambda i: (i, 0)),
            pl.BlockSpec((ROW_BLK, 1), lambda i: (i, 0)),
        ],
        out_shape=[
            jax.ShapeDtypeStruct((N_NODES, F), jnp.float32),
            jax.ShapeDtypeStruct((N_NODES, F), jnp.float32),
            jax.ShapeDtypeStruct((N_NODES, 1), jnp.float32),
        ],
    )(x, W1, degp)


def _tc2_body(p_ref, h1_ref, dinv_ref, b1_ref, w2_ref, h2_ref, g2_ref):
    dinv = dinv_ref[...]
    agg = p_ref[0, :, :] + p_ref[1, :, :]
    out1 = dinv * agg + (dinv * dinv) * h1_ref[...] + b1_ref[...]
    out1 = jnp.maximum(out1, 0.0)
    h2 = jnp.dot(out1, w2_ref[...], preferred_element_type=jnp.float32)
    h2_ref[...] = h2
    g2_ref[...] = h2 * dinv


def _tc2(p, h1, dinv, b1, W2):
    return pl.pallas_call(
        _tc2_body,
        grid=(N_BLK,),
        in_specs=[
            pl.BlockSpec((2, ROW_BLK, F), lambda i: (0, i, 0)),
            pl.BlockSpec((ROW_BLK, F), lambda i: (i, 0)),
            pl.BlockSpec((ROW_BLK, 1), lambda i: (i, 0)),
            pl.BlockSpec((1, F), lambda i: (0, 0)),
            pl.BlockSpec((F, F), lambda i: (0, 0)),
        ],
        out_specs=[
            pl.BlockSpec((ROW_BLK, F), lambda i: (i, 0)),
            pl.BlockSpec((ROW_BLK, F), lambda i: (i, 0)),
        ],
        out_shape=[
            jax.ShapeDtypeStruct((N_NODES, F), jnp.float32),
            jax.ShapeDtypeStruct((N_NODES, F), jnp.float32),
        ],
    )(p, h1, dinv, b1.reshape(1, F), W2)


def _tc3_body(p_ref, h2_ref, dinv_ref, b2_ref, wo_ref, bo_ref, out_ref):
    dinv = dinv_ref[...]
    agg = p_ref[0, :, :] + p_ref[1, :, :]
    out2 = dinv * agg + (dinv * dinv) * h2_ref[...] + b2_ref[...]
    logits = jnp.dot(out2, wo_ref[...], preferred_element_type=jnp.float32) + bo_ref[...]
    m = jnp.max(logits, axis=1, keepdims=True)
    s = jnp.log(jnp.sum(jnp.exp(logits - m), axis=1, keepdims=True))
    out_ref[...] = logits - m - s


def _tc3(p, h2, dinv, b2, Wo, bo):
    return pl.pallas_call(
        _tc3_body,
        grid=(N_BLK,),
        in_specs=[
            pl.BlockSpec((2, ROW_BLK, F), lambda i: (0, i, 0)),
            pl.BlockSpec((ROW_BLK, F), lambda i: (i, 0)),
            pl.BlockSpec((ROW_BLK, 1), lambda i: (i, 0)),
            pl.BlockSpec((1, F), lambda i: (0, 0)),
            pl.BlockSpec((F, OUT_DIM), lambda i: (0, 0)),
            pl.BlockSpec((1, OUT_DIM), lambda i: (0, 0)),
        ],
        out_specs=pl.BlockSpec((ROW_BLK, OUT_DIM), lambda i: (i, 0)),
        out_shape=jax.ShapeDtypeStruct((N_NODES, OUT_DIM), jnp.float32),
    )(p, h2, dinv, b2.reshape(1, F), Wo, bo.reshape(1, OUT_DIM))


def kernel(x, edge_index, W1, b1, W2, b2, Wo, bo):
    src = edge_index[0].astype(jnp.int32)
    dst = edge_index[1].astype(jnp.int32)
    cnt = jnp.zeros((N_NODES,), jnp.float32).at[dst].add(1.0)
    degp = jnp.stack([cnt, jnp.zeros_like(cnt)]).reshape(2, N_NODES, 1)
    h1, g1, dinv = _tc1(x, W1, degp)
    agg1 = jnp.zeros((N_NODES, F), jnp.float32).at[dst].add(g1[src])
    p1 = jnp.stack([agg1, jnp.zeros_like(agg1)])
    h2, g2 = _tc2(p1, h1, dinv, b1, W2)
    agg2 = jnp.zeros((N_NODES, F), jnp.float32).at[dst].add(g2[src])
    p2 = jnp.stack([agg2, jnp.zeros_like(agg2)])
    return _tc3(p2, h2, dinv, b2, Wo, bo)


# speedup vs baseline: 25.6011x; 8.4309x over previous
import jax
import jax.numpy as jnp
from jax import lax
from jax.experimental import pallas as pl
from jax.experimental.pallas import tpu as pltpu
from jax.experimental.pallas import tpu_sc as plsc

N_NODES = 10000
F = 128
HF = F // 2
OUT_DIM = 64
N_EDGES = 320000
ROW_BLK = 1000
N_BLK = N_NODES // ROW_BLK

NC = 2
NS = 16
CHUNK = 125
ECH = N_EDGES // CHUNK
NCH_DEG = ECH // (NC * NS)
NCH_AGG = ECH // NS
DEG_PAD = 10240

_mesh = plsc.VectorSubcoreMesh(core_axis_name="c", subcore_axis_name="s")


def _deg_body(dst2, ones_hbm, zrow_hbm, dout, didx, ones_v, stage, dacc, sem):
    c = lax.axis_index("c")
    s = lax.axis_index("s")
    cbase = (c * NS + s) * NCH_DEG
    pltpu.sync_copy(dst2.at[pl.ds(cbase, NCH_DEG)], didx)
    pltpu.sync_copy(ones_hbm, ones_v)
    pltpu.sync_copy(zrow_hbm, stage.at[pl.ds(0, 1024)])

    @pl.when(s < 10)
    def _():
        pltpu.sync_copy(stage.at[pl.ds(0, 1000)], dacc.at[pl.ds(s * 1000, 1000)])

    plsc.subcore_barrier()

    @pl.loop(0, NCH_DEG)
    def _fire(k):
        pltpu.async_copy(ones_v.at[pl.ds(0, CHUNK)], dacc.at[didx.at[k]], sem, add=True)

    @pl.loop(0, NCH_DEG)
    def _drain(k):
        pltpu.make_async_copy(ones_v.at[pl.ds(0, CHUNK)], dacc.at[didx.at[0]], sem).wait()

    plsc.subcore_barrier()

    @pl.when(s == 0)
    def _():
        pltpu.sync_copy(dacc, stage)
        pltpu.sync_copy(stage, dout.at[pl.ds(c * DEG_PAD, N_NODES)])


_deg_call = pl.kernel(
    _deg_body,
    out_type=jax.ShapeDtypeStruct((NC * DEG_PAD,), jnp.float32),
    mesh=_mesh,
    scratch_types=[
        pltpu.VMEM((NCH_DEG, CHUNK), jnp.int32),
        pltpu.VMEM((CHUNK,), jnp.float32),
        pltpu.VMEM((N_NODES,), jnp.float32),
        pltpu.VMEM_SHARED((N_NODES,), jnp.float32),
        pltpu.SemaphoreType.DMA,
    ],
)


def _agg_body(g3, src2, dst2, zeros_hbm, out_hbm, sidx, didx, rows, zbuf, acc, sem0, sem1):
    c = lax.axis_index("c")
    s = lax.axis_index("s")
    cbase = s * NCH_AGG
    pltpu.sync_copy(src2.at[pl.ds(cbase, NCH_AGG)], sidx)
    pltpu.sync_copy(dst2.at[pl.ds(cbase, NCH_AGG)], didx)
    pltpu.sync_copy(zeros_hbm, zbuf)

    @pl.loop(0, 5)
    def _zero(j):
        pltpu.sync_copy(zbuf.at[pl.ds(0, CHUNK), :],
                        acc.at[pl.ds(s * 625 + j * CHUNK, CHUNK), :])

    def prime(table):
        pltpu.make_async_copy(table.at[sidx.at[0]], rows.at[0], sem0).start()
        pltpu.make_async_copy(table.at[sidx.at[1]], rows.at[1], sem1).start()

    def run(table):
        @pl.loop(0, NCH_AGG, step=2)
        def _body(k):
            for b, sem in ((0, sem0), (1, sem1)):
                kk = k + b
                pltpu.make_async_copy(table.at[sidx.at[kk]], rows.at[b], sem).wait()
                pltpu.sync_copy(rows.at[b], acc.at[didx.at[kk]], add=True)

                @pl.when(kk + 2 < NCH_AGG)
                def _():
                    pltpu.make_async_copy(table.at[sidx.at[kk + 2]], rows.at[b], sem).start()

    @pl.when(c == 0)
    def _():
        prime(g3.at[0])

    @pl.when(c == 1)
    def _():
        prime(g3.at[1])

    plsc.subcore_barrier()

    @pl.when(c == 0)
    def _():
        run(g3.at[0])

    @pl.when(c == 1)
    def _():
        run(g3.at[1])

    plsc.subcore_barrier()

    @pl.when(s < 10)
    def _():
        @pl.loop(0, 5)
        def _out(j):
            r0 = s * 1000 + j * 200
            pltpu.sync_copy(acc.at[pl.ds(r0, 200), :], zbuf)
            pltpu.sync_copy(zbuf, out_hbm.at[c, pl.ds(r0, 200), :])


_agg_call = pl.kernel(
    _agg_body,
    out_type=jax.ShapeDtypeStruct((NC, N_NODES, HF), jnp.float32),
    mesh=_mesh,
    scratch_types=[
        pltpu.VMEM((NCH_AGG, CHUNK), jnp.int32),
        pltpu.VMEM((NCH_AGG, CHUNK), jnp.int32),
        pltpu.VMEM((2, CHUNK, HF), jnp.float32),
        pltpu.VMEM((200, HF), jnp.float32),
        pltpu.VMEM_SHARED((N_NODES, HF), jnp.float32),
        pltpu.SemaphoreType.DMA,
        pltpu.SemaphoreType.DMA,
    ],
    compiler_params=pltpu.CompilerParams(use_tc_tiling_on_sc=False),
)


def _tc1a_body(x_ref, w1_ref, h1_ref):
    h1_ref[...] = jnp.dot(x_ref[...], w1_ref[...], preferred_element_type=jnp.float32)


def _tc1a(x, W1):
    return pl.pallas_call(
        _tc1a_body,
        grid=(N_BLK,),
        in_specs=[
            pl.BlockSpec((ROW_BLK, F), lambda i: (i, 0)),
            pl.BlockSpec((F, F), lambda i: (0, 0)),
        ],
        out_specs=pl.BlockSpec((ROW_BLK, F), lambda i: (i, 0)),
        out_shape=jax.ShapeDtypeStruct((N_NODES, F), jnp.float32),
    )(x, W1)


def _tc1b_body(h1_ref, degp_ref, g3_ref, dinv_ref):
    deg = 1.0 + degp_ref[0] + degp_ref[1]
    dinv = jax.lax.rsqrt(deg)
    g = h1_ref[...] * dinv
    g3_ref[0] = g[:, :HF]
    g3_ref[1] = g[:, HF:]
    dinv_ref[...] = dinv


def _tc1b(h1, degp):
    return pl.pallas_call(
        _tc1b_body,
        grid=(N_BLK,),
        in_specs=[
            pl.BlockSpec((ROW_BLK, F), lambda i: (i, 0)),
            pl.BlockSpec((2, ROW_BLK, 1), lambda i: (0, i, 0)),
        ],
        out_specs=[
            pl.BlockSpec((2, ROW_BLK, HF), lambda i: (0, i, 0)),
            pl.BlockSpec((ROW_BLK, 1), lambda i: (i, 0)),
        ],
        out_shape=[
            jax.ShapeDtypeStruct((2, N_NODES, HF), jnp.float32),
            jax.ShapeDtypeStruct((N_NODES, 1), jnp.float32),
        ],
    )(h1, degp)


def _tc2_body(p_ref, h1_ref, dinv_ref, b1_ref, w2_ref, h2_ref, g3_ref):
    dinv = dinv_ref[...]
    agg = jnp.concatenate([p_ref[0], p_ref[1]], axis=1)
    out1 = dinv * agg + (dinv * dinv) * h1_ref[...] + b1_ref[...]
    out1 = jnp.maximum(out1, 0.0)
    h2 = jnp.dot(out1, w2_ref[...], preferred_element_type=jnp.float32)
    h2_ref[...] = h2
    g = h2 * dinv
    g3_ref[0] = g[:, :HF]
    g3_ref[1] = g[:, HF:]


def _tc2(p, h1, dinv, b1, W2):
    return pl.pallas_call(
        _tc2_body,
        grid=(N_BLK,),
        in_specs=[
            pl.BlockSpec((2, ROW_BLK, HF), lambda i: (0, i, 0)),
            pl.BlockSpec((ROW_BLK, F), lambda i: (i, 0)),
            pl.BlockSpec((ROW_BLK, 1), lambda i: (i, 0)),
            pl.BlockSpec((1, F), lambda i: (0, 0)),
            pl.BlockSpec((F, F), lambda i: (0, 0)),
        ],
        out_specs=[
            pl.BlockSpec((ROW_BLK, F), lambda i: (i, 0)),
            pl.BlockSpec((2, ROW_BLK, HF), lambda i: (0, i, 0)),
        ],
        out_shape=[
            jax.ShapeDtypeStruct((N_NODES, F), jnp.float32),
            jax.ShapeDtypeStruct((2, N_NODES, HF), jnp.float32),
        ],
    )(p, h1, dinv, b1.reshape(1, F), W2)


def _tc3_body(p_ref, h2_ref, dinv_ref, b2_ref, wo_ref, bo_ref, out_ref):
    dinv = dinv_ref[...]
    agg = jnp.concatenate([p_ref[0], p_ref[1]], axis=1)
    out2 = dinv * agg + (dinv * dinv) * h2_ref[...] + b2_ref[...]
    logits = jnp.dot(out2, wo_ref[...], preferred_element_type=jnp.float32) + bo_ref[...]
    m = jnp.max(logits, axis=1, keepdims=True)
    srow = jnp.log(jnp.sum(jnp.exp(logits - m), axis=1, keepdims=True))
    out_ref[...] = logits - m - srow


def _tc3(p, h2, dinv, b2, Wo, bo):
    return pl.pallas_call(
        _tc3_body,
        grid=(N_BLK,),
        in_specs=[
            pl.BlockSpec((2, ROW_BLK, HF), lambda i: (0, i, 0)),
            pl.BlockSpec((ROW_BLK, F), lambda i: (i, 0)),
            pl.BlockSpec((ROW_BLK, 1), lambda i: (i, 0)),
            pl.BlockSpec((1, F), lambda i: (0, 0)),
            pl.BlockSpec((F, OUT_DIM), lambda i: (0, 0)),
            pl.BlockSpec((1, OUT_DIM), lambda i: (0, 0)),
        ],
        out_specs=pl.BlockSpec((ROW_BLK, OUT_DIM), lambda i: (i, 0)),
        out_shape=jax.ShapeDtypeStruct((N_NODES, OUT_DIM), jnp.float32),
    )(p, h2, dinv, b2.reshape(1, F), Wo, bo.reshape(1, OUT_DIM))


def kernel(x, edge_index, W1, b1, W2, b2, Wo, bo):
    src2 = edge_index[0].astype(jnp.int32).reshape(ECH, CHUNK)
    dst2 = edge_index[1].astype(jnp.int32).reshape(ECH, CHUNK)
    ones_row = jnp.ones((CHUNK,), jnp.float32)
    zrow = jnp.zeros((1024,), jnp.float32)
    zeros = jnp.zeros((200, HF), jnp.float32)

    degp = _deg_call(dst2, ones_row, zrow)
    degp = degp.reshape(NC, DEG_PAD)[:, :N_NODES].reshape(NC, N_NODES, 1)
    h1 = _tc1a(x, W1)
    g1, dinv = _tc1b(h1, degp)
    p1 = _agg_call(g1, src2, dst2, zeros)
    h2, g2 = _tc2(p1, h1, dinv, b1, W2)
    p2 = _agg_call(g2, src2, dst2, zeros)
    return _tc3(p2, h2, dinv, b2, Wo, bo)


# speedup vs baseline: 26.4938x; 1.0349x over previous
import jax
import jax.numpy as jnp
from jax import lax
from jax.experimental import pallas as pl
from jax.experimental.pallas import tpu as pltpu
from jax.experimental.pallas import tpu_sc as plsc

N_NODES = 10000
F = 128
HF = F // 2
OUT_DIM = 64
N_EDGES = 320000
ROW_BLK = 1000
N_BLK = N_NODES // ROW_BLK

NC = 2
NS = 16
CHUNK = 125
ECH = N_EDGES // CHUNK
NCH_DEG = ECH // (NC * NS)
NCH_AGG = ECH // NS
DEG_PAD = 10240

_mesh = plsc.VectorSubcoreMesh(core_axis_name="c", subcore_axis_name="s")


def _deg_body(dst2, ones_hbm, zrow_hbm, dout, didx, ones_v, stage, dacc, sem):
    c = lax.axis_index("c")
    s = lax.axis_index("s")
    cbase = (c * NS + s) * NCH_DEG
    pltpu.sync_copy(dst2.at[pl.ds(cbase, NCH_DEG)], didx)
    pltpu.sync_copy(ones_hbm, ones_v)
    pltpu.sync_copy(zrow_hbm, stage.at[pl.ds(0, 1024)])

    @pl.when(s < 10)
    def _():
        pltpu.sync_copy(stage.at[pl.ds(0, 1000)], dacc.at[pl.ds(s * 1000, 1000)])

    plsc.subcore_barrier()

    @pl.loop(0, NCH_DEG)
    def _fire(k):
        pltpu.async_copy(ones_v.at[pl.ds(0, CHUNK)], dacc.at[didx.at[k]], sem, add=True)

    @pl.loop(0, NCH_DEG)
    def _drain(k):
        pltpu.make_async_copy(ones_v.at[pl.ds(0, CHUNK)], dacc.at[didx.at[0]], sem).wait()

    plsc.subcore_barrier()

    @pl.when(s == 0)
    def _():
        pltpu.sync_copy(dacc, stage)
        pltpu.sync_copy(stage, dout.at[pl.ds(c * DEG_PAD, N_NODES)])


_deg_call = pl.kernel(
    _deg_body,
    out_type=jax.ShapeDtypeStruct((NC * DEG_PAD,), jnp.float32),
    mesh=_mesh,
    scratch_types=[
        pltpu.VMEM((NCH_DEG, CHUNK), jnp.int32),
        pltpu.VMEM((CHUNK,), jnp.float32),
        pltpu.VMEM((N_NODES,), jnp.float32),
        pltpu.VMEM_SHARED((N_NODES,), jnp.float32),
        pltpu.SemaphoreType.DMA,
    ],
)


def _agg_body(g3, src2, dst2, zeros_hbm, out_hbm, sidx, didx, rows, zbuf, acc,
              g0, g1, g2, g3s, s0, s1, s2, s3):
    gsems = (g0, g1, g2, g3s)
    ssems = (s0, s1, s2, s3)
    c = lax.axis_index("c")
    s = lax.axis_index("s")
    cbase = s * NCH_AGG
    pltpu.sync_copy(src2.at[pl.ds(cbase, NCH_AGG)], sidx)
    pltpu.sync_copy(dst2.at[pl.ds(cbase, NCH_AGG)], didx)
    pltpu.sync_copy(zeros_hbm, zbuf)

    @pl.loop(0, 5)
    def _zero(j):
        pltpu.sync_copy(zbuf.at[pl.ds(0, CHUNK), :],
                        acc.at[pl.ds(s * 625 + j * CHUNK, CHUNK), :])

    def prime(table):
        pltpu.make_async_copy(table.at[sidx.at[0]], rows.at[0], gsems[0]).start()
        pltpu.make_async_copy(table.at[sidx.at[1]], rows.at[1], gsems[1]).start()

    def run(table):
        @pl.loop(0, NCH_AGG, step=4)
        def _body(k):
            for b in range(4):
                kk = k + b
                pltpu.make_async_copy(table.at[sidx.at[kk]], rows.at[b], gsems[b]).wait()
                pltpu.async_copy(rows.at[b], acc.at[didx.at[kk]], ssems[b], add=True)
                b2 = (b + 2) % 4

                @pl.when(kk >= 2)
                def _():
                    pltpu.make_async_copy(rows.at[b2], acc.at[didx.at[0]], ssems[b2]).wait()

                @pl.when(kk + 2 < NCH_AGG)
                def _():
                    pltpu.make_async_copy(table.at[sidx.at[kk + 2]], rows.at[b2], gsems[b2]).start()

        pltpu.make_async_copy(rows.at[2], acc.at[didx.at[0]], ssems[2]).wait()
        pltpu.make_async_copy(rows.at[3], acc.at[didx.at[0]], ssems[3]).wait()

    @pl.when(c == 0)
    def _():
        prime(g3.at[0])

    @pl.when(c == 1)
    def _():
        prime(g3.at[1])

    plsc.subcore_barrier()

    @pl.when(c == 0)
    def _():
        run(g3.at[0])

    @pl.when(c == 1)
    def _():
        run(g3.at[1])

    plsc.subcore_barrier()

    @pl.when(s < 10)
    def _():
        @pl.loop(0, 5)
        def _out(j):
            r0 = s * 1000 + j * 200
            pltpu.sync_copy(acc.at[pl.ds(r0, 200), :], zbuf)
            pltpu.sync_copy(zbuf, out_hbm.at[c, pl.ds(r0, 200), :])


_agg_call = pl.kernel(
    _agg_body,
    out_type=jax.ShapeDtypeStruct((NC, N_NODES, HF), jnp.float32),
    mesh=_mesh,
    scratch_types=[
        pltpu.VMEM((NCH_AGG, CHUNK), jnp.int32),
        pltpu.VMEM((NCH_AGG, CHUNK), jnp.int32),
        pltpu.VMEM((4, CHUNK, HF), jnp.float32),
        pltpu.VMEM((200, HF), jnp.float32),
        pltpu.VMEM_SHARED((N_NODES, HF), jnp.float32),
    ] + [pltpu.SemaphoreType.DMA] * 8,
    compiler_params=pltpu.CompilerParams(use_tc_tiling_on_sc=False),
)


def _tc1a_body(x_ref, w1_ref, h1_ref):
    h1_ref[...] = jnp.dot(x_ref[...], w1_ref[...], preferred_element_type=jnp.float32)


def _tc1a(x, W1):
    return pl.pallas_call(
        _tc1a_body,
        grid=(N_BLK,),
        in_specs=[
            pl.BlockSpec((ROW_BLK, F), lambda i: (i, 0)),
            pl.BlockSpec((F, F), lambda i: (0, 0)),
        ],
        out_specs=pl.BlockSpec((ROW_BLK, F), lambda i: (i, 0)),
        out_shape=jax.ShapeDtypeStruct((N_NODES, F), jnp.float32),
    )(x, W1)


def _tc1b_body(h1_ref, degp_ref, g3_ref, dinv_ref):
    deg = 1.0 + degp_ref[0] + degp_ref[1]
    dinv = jax.lax.rsqrt(deg)
    g = h1_ref[...] * dinv
    g3_ref[0] = g[:, :HF]
    g3_ref[1] = g[:, HF:]
    dinv_ref[...] = dinv


def _tc1b(h1, degp):
    return pl.pallas_call(
        _tc1b_body,
        grid=(N_BLK,),
        in_specs=[
            pl.BlockSpec((ROW_BLK, F), lambda i: (i, 0)),
            pl.BlockSpec((2, ROW_BLK, 1), lambda i: (0, i, 0)),
        ],
        out_specs=[
            pl.BlockSpec((2, ROW_BLK, HF), lambda i: (0, i, 0)),
            pl.BlockSpec((ROW_BLK, 1), lambda i: (i, 0)),
        ],
        out_shape=[
            jax.ShapeDtypeStruct((2, N_NODES, HF), jnp.float32),
            jax.ShapeDtypeStruct((N_NODES, 1), jnp.float32),
        ],
    )(h1, degp)


def _tc2_body(p_ref, h1_ref, dinv_ref, b1_ref, w2_ref, h2_ref, g3_ref):
    dinv = dinv_ref[...]
    agg = jnp.concatenate([p_ref[0], p_ref[1]], axis=1)
    out1 = dinv * agg + (dinv * dinv) * h1_ref[...] + b1_ref[...]
    out1 = jnp.maximum(out1, 0.0)
    h2 = jnp.dot(out1, w2_ref[...], preferred_element_type=jnp.float32)
    h2_ref[...] = h2
    g = h2 * dinv
    g3_ref[0] = g[:, :HF]
    g3_ref[1] = g[:, HF:]


def _tc2(p, h1, dinv, b1, W2):
    return pl.pallas_call(
        _tc2_body,
        grid=(N_BLK,),
        in_specs=[
            pl.BlockSpec((2, ROW_BLK, HF), lambda i: (0, i, 0)),
            pl.BlockSpec((ROW_BLK, F), lambda i: (i, 0)),
            pl.BlockSpec((ROW_BLK, 1), lambda i: (i, 0)),
            pl.BlockSpec((1, F), lambda i: (0, 0)),
            pl.BlockSpec((F, F), lambda i: (0, 0)),
        ],
        out_specs=[
            pl.BlockSpec((ROW_BLK, F), lambda i: (i, 0)),
            pl.BlockSpec((2, ROW_BLK, HF), lambda i: (0, i, 0)),
        ],
        out_shape=[
            jax.ShapeDtypeStruct((N_NODES, F), jnp.float32),
            jax.ShapeDtypeStruct((2, N_NODES, HF), jnp.float32),
        ],
    )(p, h1, dinv, b1.reshape(1, F), W2)


def _tc3_body(p_ref, h2_ref, dinv_ref, b2_ref, wo_ref, bo_ref, out_ref):
    dinv = dinv_ref[...]
    agg = jnp.concatenate([p_ref[0], p_ref[1]], axis=1)
    out2 = dinv * agg + (dinv * dinv) * h2_ref[...] + b2_ref[...]
    logits = jnp.dot(out2, wo_ref[...], preferred_element_type=jnp.float32) + bo_ref[...]
    m = jnp.max(logits, axis=1, keepdims=True)
    srow = jnp.log(jnp.sum(jnp.exp(logits - m), axis=1, keepdims=True))
    out_ref[...] = logits - m - srow


def _tc3(p, h2, dinv, b2, Wo, bo):
    return pl.pallas_call(
        _tc3_body,
        grid=(N_BLK,),
        in_specs=[
            pl.BlockSpec((2, ROW_BLK, HF), lambda i: (0, i, 0)),
            pl.BlockSpec((ROW_BLK, F), lambda i: (i, 0)),
            pl.BlockSpec((ROW_BLK, 1), lambda i: (i, 0)),
            pl.BlockSpec((1, F), lambda i: (0, 0)),
            pl.BlockSpec((F, OUT_DIM), lambda i: (0, 0)),
            pl.BlockSpec((1, OUT_DIM), lambda i: (0, 0)),
        ],
        out_specs=pl.BlockSpec((ROW_BLK, OUT_DIM), lambda i: (i, 0)),
        out_shape=jax.ShapeDtypeStruct((N_NODES, OUT_DIM), jnp.float32),
    )(p, h2, dinv, b2.reshape(1, F), Wo, bo.reshape(1, OUT_DIM))


def kernel(x, edge_index, W1, b1, W2, b2, Wo, bo):
    src2 = edge_index[0].astype(jnp.int32).reshape(ECH, CHUNK)
    dst2 = edge_index[1].astype(jnp.int32).reshape(ECH, CHUNK)
    ones_row = jnp.ones((CHUNK,), jnp.float32)
    zrow = jnp.zeros((1024,), jnp.float32)
    zeros = jnp.zeros((200, HF), jnp.float32)

    degp = _deg_call(dst2, ones_row, zrow)
    degp = degp.reshape(NC, DEG_PAD)[:, :N_NODES].reshape(NC, N_NODES, 1)
    h1 = _tc1a(x, W1)
    g1, dinv = _tc1b(h1, degp)
    p1 = _agg_call(g1, src2, dst2, zeros)
    h2, g2 = _tc2(p1, h1, dinv, b1, W2)
    p2 = _agg_call(g2, src2, dst2, zeros)
    return _tc3(p2, h2, dinv, b2, Wo, bo)


# speedup vs baseline: 26.9308x; 1.0165x over previous
import jax
import jax.numpy as jnp
from jax import lax
from jax.experimental import pallas as pl
from jax.experimental.pallas import tpu as pltpu
from jax.experimental.pallas import tpu_sc as plsc

N_NODES = 10000
F = 128
HF = F // 2
OUT_DIM = 64
N_EDGES = 320000
ROW_BLK = 1000
N_BLK = N_NODES // ROW_BLK

NC = 2
NS = 16
CHUNK = 125
ECH = N_EDGES // CHUNK
NCH_DEG = ECH // (NC * NS)
NCH_AGG = ECH // NS
DEG_PAD = 10240

_mesh = plsc.VectorSubcoreMesh(core_axis_name="c", subcore_axis_name="s")


def _deg_body(dst2, ones_hbm, zrow_hbm, dout, didx, ones_v, stage, dacc, sem):
    c = lax.axis_index("c")
    s = lax.axis_index("s")
    cbase = (c * NS + s) * NCH_DEG
    pltpu.sync_copy(dst2.at[pl.ds(cbase, NCH_DEG)], didx)
    pltpu.sync_copy(ones_hbm, ones_v)
    pltpu.sync_copy(zrow_hbm, stage.at[pl.ds(0, 1024)])

    @pl.when(s < 10)
    def _():
        pltpu.sync_copy(stage.at[pl.ds(0, 1000)], dacc.at[pl.ds(s * 1000, 1000)])

    plsc.subcore_barrier()

    @pl.loop(0, NCH_DEG)
    def _fire(k):
        pltpu.async_copy(ones_v.at[pl.ds(0, CHUNK)], dacc.at[didx.at[k]], sem, add=True)

    @pl.loop(0, NCH_DEG)
    def _drain(k):
        pltpu.make_async_copy(ones_v.at[pl.ds(0, CHUNK)], dacc.at[didx.at[0]], sem).wait()

    plsc.subcore_barrier()

    @pl.when(s == 0)
    def _():
        pltpu.sync_copy(dacc, stage)
        pltpu.sync_copy(stage, dout.at[pl.ds(c * DEG_PAD, N_NODES)])


_deg_call = pl.kernel(
    _deg_body,
    out_type=jax.ShapeDtypeStruct((NC * DEG_PAD,), jnp.float32),
    mesh=_mesh,
    scratch_types=[
        pltpu.VMEM((NCH_DEG, CHUNK), jnp.int32),
        pltpu.VMEM((CHUNK,), jnp.float32),
        pltpu.VMEM((N_NODES,), jnp.float32),
        pltpu.VMEM_SHARED((N_NODES,), jnp.float32),
        pltpu.SemaphoreType.DMA,
    ],
)


def _agg_body(g3, src2, dst2, zeros_hbm, out_hbm, sidx, didx, rows, zbuf, acc,
              g0, g1, g2, g3s, s0, s1, s2, s3):
    gsems = (g0, g1, g2, g3s)
    ssems = (s0, s1, s2, s3)
    c = lax.axis_index("c")
    s = lax.axis_index("s")
    cbase = s * NCH_AGG
    pltpu.sync_copy(src2.at[pl.ds(cbase, NCH_AGG)], sidx)
    pltpu.sync_copy(dst2.at[pl.ds(cbase, NCH_AGG)], didx)
    pltpu.sync_copy(zeros_hbm, zbuf)

    @pl.loop(0, 5)
    def _zero(j):
        pltpu.sync_copy(zbuf.at[pl.ds(0, CHUNK), :],
                        acc.at[pl.ds(s * 625 + j * CHUNK, CHUNK), :])

    def prime(table):
        pltpu.make_async_copy(table.at[sidx.at[0]], rows.at[0], gsems[0]).start()
        pltpu.make_async_copy(table.at[sidx.at[1]], rows.at[1], gsems[1]).start()

    def run(table):
        @pl.loop(0, NCH_AGG, step=4)
        def _body(k):
            for b in range(4):
                kk = k + b
                pltpu.make_async_copy(table.at[sidx.at[kk]], rows.at[b], gsems[b]).wait()
                pltpu.async_copy(rows.at[b], acc.at[didx.at[kk]], ssems[b], add=True)
                b2 = (b + 2) % 4

                @pl.when(kk >= 2)
                def _():
                    pltpu.make_async_copy(rows.at[b2], acc.at[didx.at[0]], ssems[b2]).wait()

                @pl.when(kk + 2 < NCH_AGG)
                def _():
                    pltpu.make_async_copy(table.at[sidx.at[kk + 2]], rows.at[b2], gsems[b2]).start()

        pltpu.make_async_copy(rows.at[2], acc.at[didx.at[0]], ssems[2]).wait()
        pltpu.make_async_copy(rows.at[3], acc.at[didx.at[0]], ssems[3]).wait()

    @pl.when(c == 0)
    def _():
        prime(g3.at[0])

    @pl.when(c == 1)
    def _():
        prime(g3.at[1])

    plsc.subcore_barrier()

    @pl.when(c == 0)
    def _():
        run(g3.at[0])

    @pl.when(c == 1)
    def _():
        run(g3.at[1])

    plsc.subcore_barrier()

    @pl.when(s < 10)
    def _():
        @pl.loop(0, 5)
        def _out(j):
            r0 = s * 1000 + j * 200
            pltpu.sync_copy(acc.at[pl.ds(r0, 200), :], zbuf)
            pltpu.sync_copy(zbuf, out_hbm.at[c, pl.ds(r0, 200), :])


_agg_call = pl.kernel(
    _agg_body,
    out_type=jax.ShapeDtypeStruct((NC, N_NODES, HF), jnp.float32),
    mesh=_mesh,
    scratch_types=[
        pltpu.VMEM((NCH_AGG, CHUNK), jnp.int32),
        pltpu.VMEM((NCH_AGG, CHUNK), jnp.int32),
        pltpu.VMEM((4, CHUNK, HF), jnp.float32),
        pltpu.VMEM((200, HF), jnp.float32),
        pltpu.VMEM_SHARED((N_NODES, HF), jnp.float32),
    ] + [pltpu.SemaphoreType.DMA] * 8,
    compiler_params=pltpu.CompilerParams(use_tc_tiling_on_sc=False),
)


def _tc1_body(x_ref, w1_ref, degp_ref, g3_ref, dinv_ref):
    deg = 1.0 + degp_ref[0] + degp_ref[1]
    dinv = jax.lax.rsqrt(deg)
    g = jnp.dot(x_ref[...], w1_ref[...], preferred_element_type=jnp.float32) * dinv
    g3_ref[0] = g[:, :HF]
    g3_ref[1] = g[:, HF:]
    dinv_ref[...] = dinv


def _tc1(x, W1, degp):
    return pl.pallas_call(
        _tc1_body,
        grid=(N_BLK,),
        in_specs=[
            pl.BlockSpec((ROW_BLK, F), lambda i: (i, 0)),
            pl.BlockSpec((F, F), lambda i: (0, 0)),
            pl.BlockSpec((2, ROW_BLK, 1), lambda i: (0, i, 0)),
        ],
        out_specs=[
            pl.BlockSpec((2, ROW_BLK, HF), lambda i: (0, i, 0)),
            pl.BlockSpec((ROW_BLK, 1), lambda i: (i, 0)),
        ],
        out_shape=[
            jax.ShapeDtypeStruct((2, N_NODES, HF), jnp.float32),
            jax.ShapeDtypeStruct((N_NODES, 1), jnp.float32),
        ],
    )(x, W1, degp)


def _tc2_body(p_ref, g1_ref, dinv_ref, b1_ref, w2_ref, g3_ref):
    dinv = dinv_ref[...]
    agg = jnp.concatenate([p_ref[0] + g1_ref[0], p_ref[1] + g1_ref[1]], axis=1)
    out1 = jnp.maximum(dinv * agg + b1_ref[...], 0.0)
    g = jnp.dot(out1, w2_ref[...], preferred_element_type=jnp.float32) * dinv
    g3_ref[0] = g[:, :HF]
    g3_ref[1] = g[:, HF:]


def _tc2(p, g1, dinv, b1, W2):
    return pl.pallas_call(
        _tc2_body,
        grid=(N_BLK,),
        in_specs=[
            pl.BlockSpec((2, ROW_BLK, HF), lambda i: (0, i, 0)),
            pl.BlockSpec((2, ROW_BLK, HF), lambda i: (0, i, 0)),
            pl.BlockSpec((ROW_BLK, 1), lambda i: (i, 0)),
            pl.BlockSpec((1, F), lambda i: (0, 0)),
            pl.BlockSpec((F, F), lambda i: (0, 0)),
        ],
        out_specs=pl.BlockSpec((2, ROW_BLK, HF), lambda i: (0, i, 0)),
        out_shape=jax.ShapeDtypeStruct((2, N_NODES, HF), jnp.float32),
    )(p, g1, dinv, b1.reshape(1, F), W2)


def _tc3_body(p_ref, g2_ref, dinv_ref, b2_ref, wo_ref, bo_ref, out_ref):
    dinv = dinv_ref[...]
    agg = jnp.concatenate([p_ref[0] + g2_ref[0], p_ref[1] + g2_ref[1]], axis=1)
    out2 = dinv * agg + b2_ref[...]
    logits = jnp.dot(out2, wo_ref[...], preferred_element_type=jnp.float32) + bo_ref[...]
    m = jnp.max(logits, axis=1, keepdims=True)
    srow = jnp.log(jnp.sum(jnp.exp(logits - m), axis=1, keepdims=True))
    out_ref[...] = logits - m - srow


def _tc3(p, g2, dinv, b2, Wo, bo):
    return pl.pallas_call(
        _tc3_body,
        grid=(N_BLK,),
        in_specs=[
            pl.BlockSpec((2, ROW_BLK, HF), lambda i: (0, i, 0)),
            pl.BlockSpec((2, ROW_BLK, HF), lambda i: (0, i, 0)),
            pl.BlockSpec((ROW_BLK, 1), lambda i: (i, 0)),
            pl.BlockSpec((1, F), lambda i: (0, 0)),
            pl.BlockSpec((F, OUT_DIM), lambda i: (0, 0)),
            pl.BlockSpec((1, OUT_DIM), lambda i: (0, 0)),
        ],
        out_specs=pl.BlockSpec((ROW_BLK, OUT_DIM), lambda i: (i, 0)),
        out_shape=jax.ShapeDtypeStruct((N_NODES, OUT_DIM), jnp.float32),
    )(p, g2, dinv, b2.reshape(1, F), Wo, bo.reshape(1, OUT_DIM))


def kernel(x, edge_index, W1, b1, W2, b2, Wo, bo):
    src2 = edge_index[0].astype(jnp.int32).reshape(ECH, CHUNK)
    dst2 = edge_index[1].astype(jnp.int32).reshape(ECH, CHUNK)
    ones_row = jnp.ones((CHUNK,), jnp.float32)
    zrow = jnp.zeros((1024,), jnp.float32)
    zeros = jnp.zeros((200, HF), jnp.float32)

    degp = _deg_call(dst2, ones_row, zrow)
    degp = degp.reshape(NC, DEG_PAD)[:, :N_NODES].reshape(NC, N_NODES, 1)
    g1, dinv = _tc1(x, W1, degp)
    p1 = _agg_call(g1, src2, dst2, zeros)
    g2 = _tc2(p1, g1, dinv, b1, W2)
    p2 = _agg_call(g2, src2, dst2, zeros)
    return _tc3(p2, g2, dinv, b2, Wo, bo)


# speedup vs baseline: 27.1648x; 1.0087x over previous
import jax
import jax.numpy as jnp
from jax import lax
from jax.experimental import pallas as pl
from jax.experimental.pallas import tpu as pltpu
from jax.experimental.pallas import tpu_sc as plsc

N_NODES = 10000
F = 128
HF = F // 2
OUT_DIM = 64
N_EDGES = 320000
ROW_BLK = 1000
N_BLK = N_NODES // ROW_BLK

NC = 2
NS = 16
CHUNK = 125
ECH = N_EDGES // CHUNK
NCH_DEG = ECH // (NC * NS)
NCH_AGG = ECH // NS
DEG_PAD = 10240

_mesh = plsc.VectorSubcoreMesh(core_axis_name="c", subcore_axis_name="s")


def _deg_body(dst2, ones_hbm, zrow_hbm, dout, didx, ones_v, stage, dacc, sem):
    c = lax.axis_index("c")
    s = lax.axis_index("s")
    cbase = (c * NS + s) * NCH_DEG
    pltpu.sync_copy(dst2.at[pl.ds(cbase, NCH_DEG)], didx)
    pltpu.sync_copy(ones_hbm, ones_v)
    pltpu.sync_copy(zrow_hbm, stage.at[pl.ds(0, 1024)])

    @pl.when(s < 10)
    def _():
        pltpu.sync_copy(stage.at[pl.ds(0, 1000)], dacc.at[pl.ds(s * 1000, 1000)])

    plsc.subcore_barrier()

    @pl.loop(0, NCH_DEG)
    def _fire(k):
        pltpu.async_copy(ones_v.at[pl.ds(0, CHUNK)], dacc.at[didx.at[k]], sem, add=True)

    @pl.loop(0, NCH_DEG)
    def _drain(k):
        pltpu.make_async_copy(ones_v.at[pl.ds(0, CHUNK)], dacc.at[didx.at[0]], sem).wait()

    plsc.subcore_barrier()

    @pl.when(s == 0)
    def _():
        pltpu.sync_copy(dacc, stage)
        pltpu.sync_copy(stage, dout.at[pl.ds(c * DEG_PAD, N_NODES)])


_deg_call = pl.kernel(
    _deg_body,
    out_type=jax.ShapeDtypeStruct((NC * DEG_PAD,), jnp.float32),
    mesh=_mesh,
    scratch_types=[
        pltpu.VMEM((NCH_DEG, CHUNK), jnp.int32),
        pltpu.VMEM((CHUNK,), jnp.float32),
        pltpu.VMEM((N_NODES,), jnp.float32),
        pltpu.VMEM_SHARED((N_NODES,), jnp.float32),
        pltpu.SemaphoreType.DMA,
    ],
)


def _agg_body(g3, src2, dst2, zeros_hbm, out_hbm, sidx, didx, rows, zbuf, acc,
              g0, g1, g2, g3s, s0, s1, s2, s3):
    gsems = (g0, g1, g2, g3s)
    ssems = (s0, s1, s2, s3)
    c = lax.axis_index("c")
    s = lax.axis_index("s")
    cbase = s * NCH_AGG
    pltpu.sync_copy(src2.at[pl.ds(cbase, NCH_AGG)], sidx)
    pltpu.sync_copy(dst2.at[pl.ds(cbase, NCH_AGG)], didx)
    pltpu.sync_copy(zeros_hbm, zbuf)

    @pl.loop(0, 5)
    def _zero(j):
        pltpu.sync_copy(zbuf.at[pl.ds(0, CHUNK), :],
                        acc.at[pl.ds(s * 625 + j * CHUNK, CHUNK), :])

    def prime(table):
        pltpu.make_async_copy(table.at[sidx.at[0]], rows.at[0], gsems[0]).start()
        pltpu.make_async_copy(table.at[sidx.at[1]], rows.at[1], gsems[1]).start()

    def run(table):
        @pl.loop(0, NCH_AGG, step=4)
        def _body(k):
            for b in range(4):
                kk = k + b
                pltpu.make_async_copy(table.at[sidx.at[kk]], rows.at[b], gsems[b]).wait()
                pltpu.async_copy(rows.at[b], acc.at[didx.at[kk]], ssems[b], add=True)
                b2 = (b + 2) % 4

                @pl.when(kk >= 2)
                def _():
                    pltpu.make_async_copy(rows.at[b2], acc.at[didx.at[0]], ssems[b2]).wait()

                @pl.when(kk + 2 < NCH_AGG)
                def _():
                    pltpu.make_async_copy(table.at[sidx.at[kk + 2]], rows.at[b2], gsems[b2]).start()

        pltpu.make_async_copy(rows.at[2], acc.at[didx.at[0]], ssems[2]).wait()
        pltpu.make_async_copy(rows.at[3], acc.at[didx.at[0]], ssems[3]).wait()

    @pl.when(c == 0)
    def _():
        prime(g3.at[0])

    @pl.when(c == 1)
    def _():
        prime(g3.at[1])

    plsc.subcore_barrier()

    @pl.when(c == 0)
    def _():
        run(g3.at[0])

    @pl.when(c == 1)
    def _():
        run(g3.at[1])

    plsc.subcore_barrier()

    @pl.loop(0, 5)
    def _out(j):
        r0 = s * 625 + j * CHUNK
        pltpu.sync_copy(acc.at[pl.ds(r0, CHUNK), :], rows.at[0])
        pltpu.sync_copy(rows.at[0], out_hbm.at[c, pl.ds(r0, CHUNK), :])


_agg_call = pl.kernel(
    _agg_body,
    out_type=jax.ShapeDtypeStruct((NC, N_NODES, HF), jnp.float32),
    mesh=_mesh,
    scratch_types=[
        pltpu.VMEM((NCH_AGG, CHUNK), jnp.int32),
        pltpu.VMEM((NCH_AGG, CHUNK), jnp.int32),
        pltpu.VMEM((4, CHUNK, HF), jnp.float32),
        pltpu.VMEM((200, HF), jnp.float32),
        pltpu.VMEM_SHARED((N_NODES, HF), jnp.float32),
    ] + [pltpu.SemaphoreType.DMA] * 8,
    compiler_params=pltpu.CompilerParams(use_tc_tiling_on_sc=False),
)


def _tc1_body(x_ref, w1_ref, degp_ref, g3_ref, dinv_ref):
    deg = 1.0 + degp_ref[0] + degp_ref[1]
    dinv = jax.lax.rsqrt(deg)
    g = jnp.dot(x_ref[...], w1_ref[...], preferred_element_type=jnp.float32) * dinv
    g3_ref[0] = g[:, :HF]
    g3_ref[1] = g[:, HF:]
    dinv_ref[...] = dinv


def _tc1(x, W1, degp):
    return pl.pallas_call(
        _tc1_body,
        grid=(N_BLK,),
        in_specs=[
            pl.BlockSpec((ROW_BLK, F), lambda i: (i, 0)),
            pl.BlockSpec((F, F), lambda i: (0, 0)),
            pl.BlockSpec((2, ROW_BLK, 1), lambda i: (0, i, 0)),
        ],
        out_specs=[
            pl.BlockSpec((2, ROW_BLK, HF), lambda i: (0, i, 0)),
            pl.BlockSpec((ROW_BLK, 1), lambda i: (i, 0)),
        ],
        out_shape=[
            jax.ShapeDtypeStruct((2, N_NODES, HF), jnp.float32),
            jax.ShapeDtypeStruct((N_NODES, 1), jnp.float32),
        ],
    )(x, W1, degp)


def _tc2_body(p_ref, g1_ref, dinv_ref, b1_ref, w2_ref, g3_ref):
    dinv = dinv_ref[...]
    agg = jnp.concatenate([p_ref[0] + g1_ref[0], p_ref[1] + g1_ref[1]], axis=1)
    out1 = jnp.maximum(dinv * agg + b1_ref[...], 0.0)
    g = jnp.dot(out1, w2_ref[...], preferred_element_type=jnp.float32) * dinv
    g3_ref[0] = g[:, :HF]
    g3_ref[1] = g[:, HF:]


def _tc2(p, g1, dinv, b1, W2):
    return pl.pallas_call(
        _tc2_body,
        grid=(N_BLK,),
        in_specs=[
            pl.BlockSpec((2, ROW_BLK, HF), lambda i: (0, i, 0)),
            pl.BlockSpec((2, ROW_BLK, HF), lambda i: (0, i, 0)),
            pl.BlockSpec((ROW_BLK, 1), lambda i: (i, 0)),
            pl.BlockSpec((1, F), lambda i: (0, 0)),
            pl.BlockSpec((F, F), lambda i: (0, 0)),
        ],
        out_specs=pl.BlockSpec((2, ROW_BLK, HF), lambda i: (0, i, 0)),
        out_shape=jax.ShapeDtypeStruct((2, N_NODES, HF), jnp.float32),
    )(p, g1, dinv, b1.reshape(1, F), W2)


def _tc3_body(p_ref, g2_ref, dinv_ref, b2_ref, wo_ref, bo_ref, out_ref):
    dinv = dinv_ref[...]
    agg = jnp.concatenate([p_ref[0] + g2_ref[0], p_ref[1] + g2_ref[1]], axis=1)
    out2 = dinv * agg + b2_ref[...]
    logits = jnp.dot(out2, wo_ref[...], preferred_element_type=jnp.float32) + bo_ref[...]
    m = jnp.max(logits, axis=1, keepdims=True)
    srow = jnp.log(jnp.sum(jnp.exp(logits - m), axis=1, keepdims=True))
    out_ref[...] = logits - m - srow


def _tc3(p, g2, dinv, b2, Wo, bo):
    return pl.pallas_call(
        _tc3_body,
        grid=(N_BLK,),
        in_specs=[
            pl.BlockSpec((2, ROW_BLK, HF), lambda i: (0, i, 0)),
            pl.BlockSpec((2, ROW_BLK, HF), lambda i: (0, i, 0)),
            pl.BlockSpec((ROW_BLK, 1), lambda i: (i, 0)),
            pl.BlockSpec((1, F), lambda i: (0, 0)),
            pl.BlockSpec((F, OUT_DIM), lambda i: (0, 0)),
            pl.BlockSpec((1, OUT_DIM), lambda i: (0, 0)),
        ],
        out_specs=pl.BlockSpec((ROW_BLK, OUT_DIM), lambda i: (i, 0)),
        out_shape=jax.ShapeDtypeStruct((N_NODES, OUT_DIM), jnp.float32),
    )(p, g2, dinv, b2.reshape(1, F), Wo, bo.reshape(1, OUT_DIM))


def kernel(x, edge_index, W1, b1, W2, b2, Wo, bo):
    src2 = edge_index[0].astype(jnp.int32).reshape(ECH, CHUNK)
    dst2 = edge_index[1].astype(jnp.int32).reshape(ECH, CHUNK)
    ones_row = jnp.ones((CHUNK,), jnp.float32)
    zrow = jnp.zeros((1024,), jnp.float32)
    zeros = jnp.zeros((200, HF), jnp.float32)

    degp = _deg_call(dst2, ones_row, zrow)
    degp = degp.reshape(NC, DEG_PAD)[:, :N_NODES].reshape(NC, N_NODES, 1)
    g1, dinv = _tc1(x, W1, degp)
    p1 = _agg_call(g1, src2, dst2, zeros)
    g2 = _tc2(p1, g1, dinv, b1, W2)
    p2 = _agg_call(g2, src2, dst2, zeros)
    return _tc3(p2, g2, dinv, b2, Wo, bo)
